# Initial kernel scaffold; baseline (speedup 1.0000x reference)
import jax
import jax.numpy as jnp
from jax.experimental import pallas as pl


def kernel(positions, c0_w0, c0_b0, c0_g0, c0_be0, c0_w1, c0_b1, c0_g1, c0_be1, c0_w2, c0_b2, c0_g2, c0_be2, c1_w0, c1_b0, c1_g0, c1_be0, c1_w1, c1_b1, c1_g1, c1_be1, c1_w2, c1_b2, c1_g2, c1_be2, c2_w0, c2_b0, c2_g0, c2_be0, c2_w1, c2_b1, c2_g1, c2_be1, c2_w2, c2_b2, c2_g2, c2_be2, p0_w, p1_w, p2_w, lin_w, lin_b):
    raise NotImplementedError("write your pallas kernel here")


# speedup vs baseline: 9.7525x; 9.7525x over previous
import functools
import math

import jax
import jax.numpy as jnp
from jax import lax
from jax.experimental import pallas as pl
from jax.experimental.pallas import tpu as pltpu
from jax.experimental.pallas import tpu_sc as plsc

B = 16
KNN = 10
KP = 16
C = 64
EPS = 1e-5
NB = 256
BIGF = 3.0e38


def _k1_body(n, d, x_cloud_ref, xt_ref, gidx_ref):
    b = pl.program_id(0)
    xa = x_cloud_ref[0]
    xbt = xt_ref[0]
    d2a = jnp.sum(xa * xa, axis=1, keepdims=True)
    d2b = jnp.sum(xbt * xbt, axis=0, keepdims=True)
    prod = lax.dot_general(xa, xbt, (((1,), (0,)), ((), ())),
                           preferred_element_type=jnp.float32)
    dist = d2a + d2b - 2.0 * prod
    row_iota = lax.broadcasted_iota(jnp.int32, (n, NB), 0)
    base = b * n
    for k in range(KNN):
        mv = jnp.min(dist, axis=0, keepdims=True)
        am = jnp.min(jnp.where(dist == mv, row_iota, n),
                     axis=0, keepdims=True)
        gidx_ref[0, k:k + 1, :] = am + base
        dist = jnp.where(row_iota == am, BIGF, dist)


def _knn(x, xt):
    _, n, d = x.shape
    grid = (B, n // NB)
    return pl.pallas_call(
        functools.partial(_k1_body, n, d),
        grid=grid,
        in_specs=[
            pl.BlockSpec((1, n, d), lambda b, j: (b, 0, 0)),
            pl.BlockSpec((1, d, NB), lambda b, j: (b, 0, j)),
        ],
        out_specs=pl.BlockSpec((1, KP, NB), lambda b, j: (b, 0, j)),
        out_shape=jax.ShapeDtypeStruct((B, KP, n), jnp.int32),
    )(x, xt)


def _sc_gather(x_flat, gidx):
    e = gidx.shape[0]
    dg = x_flat.shape[1]
    nw = 32
    ch = 128
    e_per_w = e // nw
    n_ch = e_per_w // ch
    mesh = plsc.VectorSubcoreMesh(core_axis_name="c", subcore_axis_name="s")

    @functools.partial(
        pl.kernel, mesh=mesh,
        out_type=jax.ShapeDtypeStruct((e, dg), jnp.float32),
        compiler_params=pltpu.CompilerParams(use_tc_tiling_on_sc=False),
        scratch_types=[
            pltpu.VMEM((ch,), jnp.int32),
            pltpu.VMEM((ch, dg), jnp.float32),
            pltpu.SemaphoreType.DMA,
        ],
    )
    def k(x_hbm, idx_hbm, out_hbm, idx_v, rows_v, sem):
        wid = lax.axis_index("s") * 2 + lax.axis_index("c")
        base = wid * e_per_w

        def body(i, carry):
            off = base + i * ch
            pltpu.sync_copy(idx_hbm.at[pl.ds(off, ch)], idx_v)
            pltpu.async_copy(x_hbm.at[idx_v], rows_v, sem).wait()
            pltpu.sync_copy(rows_v, out_hbm.at[pl.ds(off, ch)])
            return carry

        lax.fori_loop(0, n_ch, body, 0)

    return k(x_flat, gidx)


def _k3_body(xi_ref, xg_ref, w0_ref, b0_ref, h1_ref, st_ref):
    xi = xi_ref[0]
    xj = xg_ref[0]
    dg = xi.shape[-1]
    xib = jnp.broadcast_to(xi[None], xj.shape)
    msg = jnp.concatenate([xib, xj - xib], axis=2)
    mf = msg.reshape(KNN * NB, 2 * dg)
    h = jnp.maximum(
        lax.dot_general(mf, w0_ref[...], (((1,), (0,)), ((), ())),
                        preferred_element_type=jnp.float32)
        + b0_ref[...], 0.0)
    h1_ref[0] = h.reshape(KNN, NB, C)
    s = jnp.sum(h, axis=0, keepdims=True)
    q = jnp.sum(h * h, axis=0, keepdims=True)
    @pl.when(jnp.logical_and(pl.program_id(0) == 0, pl.program_id(1) == 0))
    def _():
        st_ref[...] = jnp.zeros_like(st_ref)
    st_ref[0:1, 0:C] += s
    st_ref[1:2, 0:C] += q


def _edge_pass1(xpad, xg, w0p, b0):
    _, _, n, dg = xg.shape
    grid = (B, n // NB)
    small = lambda b, j: (0, 0)
    return pl.pallas_call(
        _k3_body,
        grid=grid,
        in_specs=[
            pl.BlockSpec((1, NB, dg), lambda b, j: (b, j, 0)),
            pl.BlockSpec((1, KNN, NB, dg), lambda b, j: (b, 0, j, 0)),
            pl.BlockSpec((2 * dg, C), small),
            pl.BlockSpec((1, C), small),
        ],
        out_specs=[
            pl.BlockSpec((1, KNN, NB, C), lambda b, j: (b, 0, j, 0)),
            pl.BlockSpec((8, 128), small),
        ],
        out_shape=[
            jax.ShapeDtypeStruct((B, KNN, n, C), jnp.float32),
            jax.ShapeDtypeStruct((8, 128), jnp.float32),
        ],
    )(xpad, xg, w0p, b0)


def _k4_body(reduce_k, h_ref, m_ref, sq_ref, g_ref, be_ref, w_ref, bb_ref,
             *out_refs):
    h = h_ref[0]
    hb = g_ref[...] * (h - m_ref[...]) / sq_ref[...] + be_ref[...]
    hf = hb.reshape(KNN * NB, C)
    h2 = jnp.maximum(
        lax.dot_general(hf, w_ref[...], (((1,), (0,)), ((), ())),
                        preferred_element_type=jnp.float32)
        + bb_ref[...], 0.0)
    s = jnp.sum(h2, axis=0, keepdims=True)
    q = jnp.sum(h2 * h2, axis=0, keepdims=True)
    if reduce_k:
        mx_ref, mn_ref, st_ref = out_refs
        h2r = h2.reshape(KNN, NB, C)
        mx_ref[0] = jnp.max(h2r, axis=0)
        mn_ref[0] = jnp.min(h2r, axis=0)
    else:
        h2_ref, st_ref = out_refs
        h2_ref[0] = h2.reshape(KNN, NB, C)
    @pl.when(jnp.logical_and(pl.program_id(0) == 0, pl.program_id(1) == 0))
    def _():
        st_ref[...] = jnp.zeros_like(st_ref)
    st_ref[0:1, 0:C] += s
    st_ref[1:2, 0:C] += q


def _edge_pass(h_in, m, sq, g, be, w, bb, reduce_k):
    _, _, n, _ = h_in.shape
    grid = (B, n // NB)
    small = lambda b, j: (0, 0)
    in_specs = [
        pl.BlockSpec((1, KNN, NB, C), lambda b, j: (b, 0, j, 0)),
        pl.BlockSpec((1, C), small),
        pl.BlockSpec((1, C), small),
        pl.BlockSpec((1, C), small),
        pl.BlockSpec((1, C), small),
        pl.BlockSpec((C, C), small),
        pl.BlockSpec((1, C), small),
    ]
    if reduce_k:
        out_specs = [
            pl.BlockSpec((1, NB, C), lambda b, j: (b, j, 0)),
            pl.BlockSpec((1, NB, C), lambda b, j: (b, j, 0)),
            pl.BlockSpec((8, 128), small),
        ]
        out_shape = [
            jax.ShapeDtypeStruct((B, n, C), jnp.float32),
            jax.ShapeDtypeStruct((B, n, C), jnp.float32),
            jax.ShapeDtypeStruct((8, 128), jnp.float32),
        ]
    else:
        out_specs = [
            pl.BlockSpec((1, KNN, NB, C), lambda b, j: (b, 0, j, 0)),
            pl.BlockSpec((8, 128), small),
        ]
        out_shape = [
            jax.ShapeDtypeStruct((B, KNN, n, C), jnp.float32),
            jax.ShapeDtypeStruct((8, 128), jnp.float32),
        ]
    return pl.pallas_call(
        functools.partial(_k4_body, reduce_k),
        grid=grid,
        in_specs=in_specs,
        out_specs=out_specs,
        out_shape=out_shape,
    )(h_in, m, sq, g, be, w, bb)


def _k6_body(n, nk, mx_ref, mn_ref, m_ref, sq_ref, g_ref, be_ref, w_ref,
             nrm_ref, xn_ref, feat_ref):
    m3 = m_ref[...]
    sq3 = sq_ref[...]
    g3 = g_ref[...]
    be3 = be_ref[...]
    hm = jnp.where(g3 > 0.0, mx_ref[0], mn_ref[0])
    y = g3 * (hm - m3) / sq3 + be3
    sc = jnp.tanh(
        lax.dot_general(y, w_ref[...], (((1,), (0,)), ((), ())),
                        preferred_element_type=jnp.float32)
        / nrm_ref[0, 0])
    cb = NB
    nch = n // cb
    s_row = jnp.zeros((1, n), jnp.float32)
    for ci in range(nch):
        io0 = lax.broadcasted_iota(jnp.int32, (cb, n), 0) + ci * cb
        io1 = lax.broadcasted_iota(jnp.int32, (cb, n), 1)
        scc = sc[ci * cb:(ci + 1) * cb, :]
        s_row = s_row + jnp.sum(jnp.where(io0 == io1, scc, 0.0),
                                axis=0, keepdims=True)
    rank_row = jnp.zeros((1, n), jnp.int32)
    feat = jnp.full((1, C), -BIGF)
    z_parts = []
    for ci in range(nch):
        io0 = lax.broadcasted_iota(jnp.int32, (cb, n), 0) + ci * cb
        io1 = lax.broadcasted_iota(jnp.int32, (cb, n), 1)
        scc = sc[ci * cb:(ci + 1) * cb, :]
        gt = s_row > scc
        eqlt = jnp.logical_and(s_row == scc, io1 < io0)
        cnt = jnp.sum(jnp.logical_or(gt, eqlt).astype(jnp.int32),
                      axis=1, keepdims=True)
        rank_row = rank_row + jnp.sum(
            jnp.where(io0 == io1, cnt, 0), axis=0, keepdims=True)
        zc = y[ci * cb:(ci + 1) * cb, :] * scc
        z_parts.append(zc)
        zm = jnp.where(cnt < nk, zc, -BIGF)
        feat = jnp.maximum(feat, jnp.max(zm, axis=0, keepdims=True))
    z = jnp.concatenate(z_parts, axis=0)
    feat_ref[0] = feat
    rc = NB
    for ci in range(nk // rc):
        ohc = jnp.where(
            lax.broadcasted_iota(jnp.int32, (rc, n), 0) + (ci * rc)
            == rank_row, 1.0, 0.0)
        xn_ref[0, ci * rc:(ci + 1) * rc, :] = lax.dot_general(
            ohc, z, (((1,), (0,)), ((), ())),
            preferred_element_type=jnp.float32,
            precision=lax.Precision.HIGHEST)


def _pool(mx, mn, m, sq, g, be, w, nrm):
    _, n, _ = mx.shape
    nk = n // 2
    small = lambda b: (0, 0)
    return pl.pallas_call(
        functools.partial(_k6_body, n, nk),
        grid=(B,),
        in_specs=[
            pl.BlockSpec((1, n, C), lambda b: (b, 0, 0)),
            pl.BlockSpec((1, n, C), lambda b: (b, 0, 0)),
            pl.BlockSpec((1, C), small),
            pl.BlockSpec((1, C), small),
            pl.BlockSpec((1, C), small),
            pl.BlockSpec((1, C), small),
            pl.BlockSpec((C, 1), small),
            pl.BlockSpec((1, 1), small),
        ],
        out_specs=[
            pl.BlockSpec((1, nk, C), lambda b: (b, 0, 0)),
            pl.BlockSpec((1, 1, C), lambda b: (b, 0, 0)),
        ],
        out_shape=[
            jax.ShapeDtypeStruct((B, nk, C), jnp.float32),
            jax.ShapeDtypeStruct((B, 1, C), jnp.float32),
        ],
    )(mx, mn, m, sq, g, be, w, nrm)


def _k7_body(f_ref, w_ref, b_ref, o_ref):
    f = f_ref[...].reshape(B, 3 * C)
    o_ref[...] = lax.dot_general(f, w_ref[...], (((1,), (0,)), ((), ())),
                                 preferred_element_type=jnp.float32) \
        + b_ref[...]


def _final(feats, lin_w, lin_b):
    f = jnp.concatenate(feats, axis=-1)
    return pl.pallas_call(
        _k7_body,
        in_specs=[
            pl.BlockSpec((B, 1, 3 * C), lambda: (0, 0, 0)),
            pl.BlockSpec((3 * C, 128), lambda: (0, 0)),
            pl.BlockSpec((1, 128), lambda: (0, 0)),
        ],
        out_specs=pl.BlockSpec((B, 128), lambda: (0, 0)),
        out_shape=jax.ShapeDtypeStruct((B, 128), jnp.float32),
    )(f, lin_w, lin_b.reshape(1, 128))


def _bn_params(st, e):
    m = st[0, :C] / e
    v = st[1, :C] / e - m * m
    sq = jnp.sqrt(v + EPS)
    return m.reshape(1, C), sq.reshape(1, C)


def kernel(positions, c0_w0, c0_b0, c0_g0, c0_be0, c0_w1, c0_b1, c0_g1, c0_be1,
           c0_w2, c0_b2, c0_g2, c0_be2, c1_w0, c1_b0, c1_g0, c1_be0, c1_w1,
           c1_b1, c1_g1, c1_be1, c1_w2, c1_b2, c1_g2, c1_be2, c2_w0, c2_b0,
           c2_g0, c2_be0, c2_w1, c2_b1, c2_g1, c2_be1, c2_w2, c2_b2, c2_g2,
           c2_be2, p0_w, p1_w, p2_w, lin_w, lin_b):
    params = [
        (c0_w0, c0_b0, c0_g0, c0_be0, c0_w1, c0_b1, c0_g1, c0_be1,
         c0_w2, c0_b2, c0_g2, c0_be2, p0_w),
        (c1_w0, c1_b0, c1_g0, c1_be0, c1_w1, c1_b1, c1_g1, c1_be1,
         c1_w2, c1_b2, c1_g2, c1_be2, p1_w),
        (c2_w0, c2_b0, c2_g0, c2_be0, c2_w1, c2_b1, c2_g1, c2_be1,
         c2_w2, c2_b2, c2_g2, c2_be2, p2_w),
    ]
    x = positions
    feats = []
    for l in range(3):
        (w0, b0, g0, be0, w1, b1, g1, be1, w2, b2, g2, be2, pw) = params[l]
        n, d = x.shape[1], x.shape[2]
        e = B * KNN * n
        dg = 16 if d < 16 else d
        if dg != d:
            xpad = jnp.pad(x, ((0, 0), (0, 0), (0, dg - d)))
            w0p = jnp.zeros((2 * dg, C), jnp.float32)
            w0p = w0p.at[:d].set(w0[:d]).at[dg:dg + d].set(w0[d:])
        else:
            xpad = x
            w0p = w0
        xt = jnp.swapaxes(x, 1, 2)
        gidx16 = _knn(x, xt)
        gidx = gidx16[:, :KNN, :].reshape(-1)
        xg = _sc_gather(xpad.reshape(-1, dg), gidx).reshape(B, KNN, n, dg)
        h1, st1 = _edge_pass1(xpad, xg, w0p, b0.reshape(1, C))
        m1, sq1 = _bn_params(st1, e)
        h2, st2 = _edge_pass(h1, m1, sq1, g0.reshape(1, C), be0.reshape(1, C),
                             w1, b1.reshape(1, C), False)
        m2, sq2 = _bn_params(st2, e)
        mx, mn, st3 = _edge_pass(h2, m2, sq2, g1.reshape(1, C),
                                 be1.reshape(1, C), w2, b2.reshape(1, C), True)
        m3, sq3 = _bn_params(st3, e)
        nrm = jnp.sqrt(jnp.sum(pw * pw)).reshape(1, 1)
        x, feat = _pool(mx, mn, m3, sq3, g2.reshape(1, C), be2.reshape(1, C),
                        pw.reshape(C, 1), nrm)
        feats.append(feat)
    return _final(feats, lin_w, lin_b)
